# Initial kernel scaffold; baseline (speedup 1.0000x reference)
import jax
import jax.numpy as jnp
from jax.experimental import pallas as pl


def kernel(x, W_emb, W_lin, lin_bias):
    raise NotImplementedError("write your pallas kernel here")


# speedup vs baseline: 1.0613x; 1.0613x over previous
import functools

import jax
import jax.numpy as jnp
import numpy as np
from jax import lax
from jax.experimental import pallas as pl
from jax.experimental.pallas import tpu as pltpu
from jax.experimental.pallas import tpu_sc as plsc

FIELD = 38462
B = 16384
F = 26
D = 16
NC, NS = 2, 16
NW = NC * NS
SPW = B // NW
C = 64
NCHUNK = SPW // C
ROWS = C * F
GL = 128
G = ROWS // GL
GP = 16
IDX_ROWS = NW * NCHUNK * GP


_GATHER_DNUMS = lax.GatherDimensionNumbers(
    offset_dims=(), collapsed_slice_dims=(0,), start_index_map=(0,))


def _shuffle(v, idx):
    return lax.gather(v, idx[:, None], _GATHER_DNUMS, slice_sizes=(1,),
                      mode=lax.GatherScatterMode.PROMISE_IN_BOUNDS)


def _lane_sum(v):
    iota = lax.iota(jnp.int32, D)
    for k in (8, 4, 2, 1):
        idx = lax.bitwise_and(iota + k, D - 1)
        v = v + _shuffle(v, idx)
    return v


def _fm_body(lx_hbm, fx_hbm, w_emb, w_lin, bias_hbm, out_hbm,
             idx_lin, idx_emb, rows_lin, rows_emb, outv, biasv, sem):
    wid = lax.axis_index("s") * NC + lax.axis_index("c")
    pltpu.sync_copy(bias_hbm, biasv)
    bias = biasv[...]

    def chunk_body(c, carry):
        base_row = (wid * NCHUNK + c) * GP
        pltpu.sync_copy(lx_hbm.at[pl.ds(base_row, GP)], idx_lin)
        pltpu.sync_copy(fx_hbm.at[pl.ds(base_row, GP)], idx_emb)
        cps = []
        for j in range(G):
            cps.append(pltpu.async_copy(
                w_lin.at[idx_lin.at[j]], rows_lin.at[pl.ds(j * GL, GL)], sem))
            cps.append(pltpu.async_copy(
                w_emb.at[idx_emb.at[j]], rows_emb.at[pl.ds(j * GL, GL)], sem))
        for cp in cps:
            cp.wait()

        def samp(i, carry2):
            r0 = i * F
            lin = rows_lin[r0]
            e = rows_emb[r0]
            s = e
            q = e * e
            for f in range(1, F):
                lin = lin + rows_lin[r0 + f]
                e = rows_emb[r0 + f]
                s = s + e
                q = q + e * e
            fm = _lane_sum(s * s - q)
            res = lin + bias + 0.5 * fm
            outv[i] = 1.0 / (1.0 + jnp.exp(-res))
            return carry2

        lax.fori_loop(0, C, samp, 0)
        out_base = wid * SPW + c * C
        pltpu.sync_copy(outv, out_hbm.at[pl.ds(out_base, C)])
        return carry

    lax.fori_loop(0, NCHUNK, chunk_body, 0)


_mesh = plsc.VectorSubcoreMesh(
    core_axis_name="c", subcore_axis_name="s", num_cores=NC, num_subcores=NS)

_fm_kernel = functools.partial(
    pl.kernel,
    out_type=jax.ShapeDtypeStruct((B, D), jnp.float32),
    mesh=_mesh,
    scratch_types=[
        pltpu.VMEM((GP, GL), jnp.int32),
        pltpu.VMEM((GP, GL), jnp.int32),
        pltpu.VMEM((ROWS, D), jnp.float32),
        pltpu.VMEM((ROWS, D), jnp.float32),
        pltpu.VMEM((C, D), jnp.float32),
        pltpu.VMEM((D,), jnp.float32),
        pltpu.SemaphoreType.DMA,
    ],
    compiler_params=pltpu.CompilerParams(use_tc_tiling_on_sc=False),
)(_fm_body)


def _pad_chunks(flat):
    blocks = flat.reshape(NW * NCHUNK, ROWS)
    blocks = jnp.pad(blocks, ((0, 0), (0, GP * GL - ROWS)))
    return blocks.reshape(IDX_ROWS, GL)


def kernel(x, W_emb, W_lin, lin_bias):
    x = x.astype(jnp.int32)
    offsets = jnp.asarray(np.arange(F, dtype=np.int32) * FIELD)
    lx = _pad_chunks((x + offsets[None, :]).reshape(-1))
    fx = _pad_chunks(x.reshape(-1))
    bias16 = jnp.broadcast_to(lin_bias.astype(jnp.float32), (D,))
    return _fm_kernel(lx, fx, W_emb, W_lin, bias16)


# speedup vs baseline: 1.0998x; 1.0363x over previous
import functools

import jax
import jax.numpy as jnp
import numpy as np
from jax import lax
from jax.experimental import pallas as pl
from jax.experimental.pallas import tpu as pltpu
from jax.experimental.pallas import tpu_sc as plsc

FIELD = 38462
B = 16384
F = 26
D = 16
NC, NS = 2, 16
NW = NC * NS
SPW = B // NW
C = 64
NCHUNK = SPW // C
ROWS = C * F
GL = 128
G = ROWS // GL
GP = 16
IDX_ROWS = NW * NCHUNK * GP

_GATHER_DNUMS = lax.GatherDimensionNumbers(
    offset_dims=(), collapsed_slice_dims=(0,), start_index_map=(0,))


def _shuffle(v, idx):
    return lax.gather(v, idx[:, None], _GATHER_DNUMS, slice_sizes=(1,),
                      mode=lax.GatherScatterMode.PROMISE_IN_BOUNDS)


def _lane_sum(v):
    iota = lax.iota(jnp.int32, D)
    for k in (8, 4, 2, 1):
        idx = lax.bitwise_and(iota + k, D - 1)
        v = v + _shuffle(v, idx)
    return v


def _fm_body(lx_hbm, fx_hbm, w_emb, w_lin, bias_hbm, out_hbm,
             idx_lin, idx_emb, rows_lin, rows_emb, outv, biasv, sem0, sem1):
    wid = lax.axis_index("s") * NC + lax.axis_index("c")
    sems = (sem0, sem1)
    pltpu.sync_copy(bias_hbm, biasv)
    bias = biasv[...]

    def gathers(c, b):
        cps = []
        for j in range(G):
            cps.append(pltpu.make_async_copy(
                w_lin.at[idx_lin.at[b].at[j]],
                rows_lin.at[b].at[pl.ds(j * GL, GL)], sems[b]))
            cps.append(pltpu.make_async_copy(
                w_emb.at[idx_emb.at[b].at[j]],
                rows_emb.at[b].at[pl.ds(j * GL, GL)], sems[b]))
        return cps

    def issue(c, b):
        base_row = (wid * NCHUNK + c) * GP
        pltpu.sync_copy(lx_hbm.at[pl.ds(base_row, GP)], idx_lin.at[b])
        pltpu.sync_copy(fx_hbm.at[pl.ds(base_row, GP)], idx_emb.at[b])
        for cp in gathers(c, b):
            cp.start()

    def wait(c, b):
        for cp in gathers(c, b):
            cp.wait()

    def compute(c, b):
        rl = rows_lin.at[b]
        re = rows_emb.at[b]

        def samp(i, carry2):
            r0 = i * F
            lin = rl[r0]
            e = re[r0]
            s = e
            q = e * e
            for f in range(1, F):
                lin = lin + rl[r0 + f]
                e = re[r0 + f]
                s = s + e
                q = q + e * e
            fm = _lane_sum(s * s - q)
            res = lin + bias + 0.5 * fm
            outv[i] = 1.0 / (1.0 + jnp.exp(-res))
            return carry2

        lax.fori_loop(0, C, samp, 0)
        out_base = wid * SPW + c * C
        pltpu.sync_copy(outv, out_hbm.at[pl.ds(out_base, C)])

    issue(0, 0)

    def pair_body(k, carry):
        c0 = 2 * k
        issue(c0 + 1, 1)
        wait(c0, 0)
        compute(c0, 0)

        @pl.when(c0 + 2 < NCHUNK)
        def _():
            issue(c0 + 2, 0)

        wait(c0 + 1, 1)
        compute(c0 + 1, 1)
        return carry

    lax.fori_loop(0, NCHUNK // 2, pair_body, 0)


_mesh = plsc.VectorSubcoreMesh(
    core_axis_name="c", subcore_axis_name="s", num_cores=NC, num_subcores=NS)

_fm_kernel = functools.partial(
    pl.kernel,
    out_type=jax.ShapeDtypeStruct((B, D), jnp.float32),
    mesh=_mesh,
    scratch_types=[
        pltpu.VMEM((2, GP, GL), jnp.int32),
        pltpu.VMEM((2, GP, GL), jnp.int32),
        pltpu.VMEM((2, ROWS, D), jnp.float32),
        pltpu.VMEM((2, ROWS, D), jnp.float32),
        pltpu.VMEM((C, D), jnp.float32),
        pltpu.VMEM((D,), jnp.float32),
        pltpu.SemaphoreType.DMA,
        pltpu.SemaphoreType.DMA,
    ],
    compiler_params=pltpu.CompilerParams(use_tc_tiling_on_sc=False),
)(_fm_body)


def _pad_chunks(flat):
    blocks = flat.reshape(NW * NCHUNK, ROWS)
    blocks = jnp.pad(blocks, ((0, 0), (0, GP * GL - ROWS)))
    return blocks.reshape(IDX_ROWS, GL)


def kernel(x, W_emb, W_lin, lin_bias):
    x = x.astype(jnp.int32)
    offsets = jnp.asarray(np.arange(F, dtype=np.int32) * FIELD)
    lx = _pad_chunks((x + offsets[None, :]).reshape(-1))
    fx = _pad_chunks(x.reshape(-1))
    bias16 = jnp.broadcast_to(lin_bias.astype(jnp.float32), (D,))
    return _fm_kernel(lx, fx, W_emb, W_lin, bias16)


# speedup vs baseline: 1.8009x; 1.6375x over previous
import functools

import jax
import jax.numpy as jnp
import numpy as np
from jax import lax
from jax.experimental import pallas as pl
from jax.experimental.pallas import tpu as pltpu
from jax.experimental.pallas import tpu_sc as plsc

FIELD = 38462
B = 16384
F = 26
D = 16
NC, NS = 2, 16
NW = NC * NS
SPW = B // NW
C = 64
NCHUNK = SPW // C
ROWS = C * F
GL = 128
G = ROWS // GL
GP = 16
IDX_ROWS = NW * NCHUNK * GP
EMB_ROWS = 38464

_GATHER_DNUMS = lax.GatherDimensionNumbers(
    offset_dims=(), collapsed_slice_dims=(0,), start_index_map=(0,))


def _shuffle(v, idx):
    return lax.gather(v, idx[:, None], _GATHER_DNUMS, slice_sizes=(1,),
                      mode=lax.GatherScatterMode.PROMISE_IN_BOUNDS)


def _lane_sum(v):
    iota = lax.iota(jnp.int32, D)
    for k in (8, 4, 2, 1):
        idx = lax.bitwise_and(iota + k, D - 1)
        v = v + _shuffle(v, idx)
    return v


def _fm_body(lx_hbm, fx_hbm, w_emb, w_lin, bias_hbm, out_hbm,
             idx_lin, idx_emb, rows_lin, rows_emb, outv, biasv, sem0, sem1):
    wid = lax.axis_index("s") * NC + lax.axis_index("c")
    sems = (sem0, sem1)
    pltpu.sync_copy(bias_hbm, biasv)
    bias = biasv[...]

    def gathers(c, b):
        cps = []
        for j in range(G):
            cps.append(pltpu.make_async_copy(
                w_lin.at[idx_lin.at[b].at[j]],
                rows_lin.at[b].at[pl.ds(j * GL, GL)], sems[b]))
            cps.append(pltpu.make_async_copy(
                w_emb.at[idx_emb.at[b].at[j]],
                rows_emb.at[b].at[pl.ds(j * GL, GL)], sems[b]))
        return cps

    def issue(c, b):
        base_row = (wid * NCHUNK + c) * GP
        pltpu.sync_copy(lx_hbm.at[pl.ds(base_row, GP)], idx_lin.at[b])
        pltpu.sync_copy(fx_hbm.at[pl.ds(base_row, GP)], idx_emb.at[b])
        for cp in gathers(c, b):
            cp.start()

    def wait(c, b):
        for cp in gathers(c, b):
            cp.wait()

    def compute(c, b):
        rl = rows_lin.at[b]
        re = rows_emb.at[b]

        def samp(i, carry2):
            r0 = i * F
            lin = rl[r0]
            e = re[r0]
            s = e
            q = e * e
            for f in range(1, F):
                lin = lin + rl[r0 + f]
                e = re[r0 + f]
                s = s + e
                q = q + e * e
            fm = _lane_sum(s * s - q)
            res = lin + bias + 0.5 * fm
            outv[i] = 1.0 / (1.0 + jnp.exp(-res))
            return carry2

        lax.fori_loop(0, C, samp, 0)
        out_base = wid * SPW + c * C
        pltpu.sync_copy(outv, out_hbm.at[pl.ds(out_base, C)])

    issue(0, 0)

    def pair_body(k, carry):
        c0 = 2 * k
        issue(c0 + 1, 1)
        wait(c0, 0)
        compute(c0, 0)

        @pl.when(c0 + 2 < NCHUNK)
        def _():
            issue(c0 + 2, 0)

        wait(c0 + 1, 1)
        compute(c0 + 1, 1)
        return carry

    lax.fori_loop(0, NCHUNK // 2, pair_body, 0)


_mesh = plsc.VectorSubcoreMesh(
    core_axis_name="c", subcore_axis_name="s", num_cores=NC, num_subcores=NS)

_fm_kernel = functools.partial(
    pl.kernel,
    out_type=jax.ShapeDtypeStruct((B, D), jnp.float32),
    mesh=_mesh,
    scratch_types=[
        pltpu.VMEM((2, GP, GL), jnp.int32),
        pltpu.VMEM((2, GP, GL), jnp.int32),
        pltpu.VMEM((2, ROWS, D), jnp.float32),
        pltpu.VMEM((2, ROWS, D), jnp.float32),
        pltpu.VMEM((C, D), jnp.float32),
        pltpu.VMEM((D,), jnp.float32),
        pltpu.SemaphoreType.DMA,
        pltpu.SemaphoreType.DMA,
    ],
    compiler_params=pltpu.CompilerParams(use_tc_tiling_on_sc=False),
)(_fm_body)


def _pad_chunks(flat):
    blocks = flat.reshape(NW * NCHUNK, ROWS)
    blocks = jnp.pad(blocks, ((0, 0), (0, GP * GL - ROWS)))
    return blocks.reshape(IDX_ROWS, GL)


def kernel(x, W_emb, W_lin, lin_bias):
    x = x.astype(jnp.int32)
    offsets = jnp.asarray(np.arange(F, dtype=np.int32) * FIELD)
    lx = _pad_chunks((x + offsets[None, :]).reshape(-1))
    fx = _pad_chunks(x.reshape(-1))
    bias16 = jnp.broadcast_to(lin_bias.astype(jnp.float32), (D,))
    w_emb_s = lax.slice(W_emb, (0, 0), (EMB_ROWS, D))
    return _fm_kernel(lx, fx, w_emb_s, W_lin, bias16)
